# Initial kernel scaffold; baseline (speedup 1.0000x reference)
import jax
import jax.numpy as jnp
from jax.experimental import pallas as pl


def kernel(pred_logits, pred_boxes, positive_map, target_sizes):
    raise NotImplementedError("write your pallas kernel here")


# speedup vs baseline: 1.7466x; 1.7466x over previous
import jax
import jax.numpy as jnp
from jax import lax
from jax.experimental import pallas as pl
from jax.experimental.pallas import tpu as pltpu

_B, _Q, _T, _C, _K = 16, 900, 256, 80, 300
_CP = 128
_KP = 384
_N = _Q * _C
_ONE_BITS = 0x3F800001


def _iotaf(shape, dim):
    return lax.broadcasted_iota(jnp.int32, shape, dim).astype(jnp.float32)


def _postprocess_kernel(logits_ref, boxes_ref, pmap_ref, ts_ref,
                        scores_ref, labels_ref, boxes_out_ref):
    f32 = jnp.float32

    pm = pmap_ref[...]
    sums = jnp.sum(pm, axis=1, keepdims=True)
    safe = jnp.where(sums == 0.0, 1.0, sums)
    pmn = jnp.where(sums != 0.0, pm / safe, pm)
    pmnp = jnp.concatenate(
        [pmn, jnp.zeros((_CP - _C, _T), f32)], axis=0)

    sig = jax.nn.sigmoid(logits_ref[...].reshape(_Q, _T))
    prob = lax.dot_general(sig, pmnp, (((1,), (1,)), ((), ())),
                           preferred_element_type=f32)
    lane = lax.broadcasted_iota(jnp.int32, (_Q, _CP), 1)
    row = lax.broadcasted_iota(jnp.int32, (_Q, _CP), 0)
    w = jnp.where(lane < _C, prob, -1.0)
    fi = row * _C + lane

    def bis1(_, lohi):
        lo, hi = lohi
        mid = lo + (hi - lo) // 2
        t = lax.bitcast_convert_type(mid, f32)
        cnt = jnp.sum((w >= t).astype(f32))
        big = cnt >= float(_K)
        return jnp.where(big, mid, lo), jnp.where(big, hi, mid)

    lo, _hi = lax.fori_loop(0, 31, bis1, (jnp.int32(0), jnp.int32(_ONE_BITS)))
    vk = lax.bitcast_convert_type(lo, f32)

    m = jnp.sum((w > vk).astype(f32))
    r = float(_K) - m

    def bis2(_, lohi):
        lo2, hi2 = lohi
        mid2 = lo2 + (hi2 - lo2) // 2
        cnt2 = jnp.sum(((w == vk) & (fi < mid2)).astype(f32))
        big = cnt2 >= r
        return jnp.where(big, lo2, mid2), jnp.where(big, mid2, hi2)

    _lo2, j = lax.fori_loop(0, 17, bis2, (jnp.int32(0), jnp.int32(_N)))

    sel = (w > vk) | ((w == vk) & (fi < j))
    self_ = sel.astype(f32)

    s = jnp.sum(self_, axis=1, keepdims=True)
    tri = (_iotaf((_Q, _Q), 0) >
           _iotaf((_Q, _Q), 1)).astype(f32)
    rpre = lax.dot_general(tri, s, (((1,), (0,)), ((), ())),
                           preferred_element_type=f32,
                           precision=lax.Precision.HIGHEST)
    pio = _iotaf((_Q, _KP), 1)
    oh = ((rpre <= pio) & (pio < rpre + s)).astype(f32)
    g = lax.dot_general(oh, w, (((0,), (0,)), ((), ())),
                        preferred_element_type=f32,
                           precision=lax.Precision.HIGHEST)
    qcol = _iotaf((_Q, 1), 0)
    qofp = lax.dot_general(oh, qcol, (((0,), (0,)), ((), ())),
                           preferred_element_type=f32,
                           precision=lax.Precision.HIGHEST)
    rofp = lax.dot_general(oh, rpre, (((0,), (0,)), ((), ())),
                           preferred_element_type=f32,
                           precision=lax.Precision.HIGHEST)
    pcol = _iotaf((_KP, 1), 0)
    op = pcol - rofp

    lane2 = _iotaf((_KP, _CP), 1)
    fi2 = qofp * float(_C) + lane2
    jf = j.astype(f32)
    sel2 = (g > vk) | ((g == vk) & (fi2 < jf))
    sel2f = sel2.astype(f32)
    triu = (_iotaf((_CP, _CP), 0) <
            _iotaf((_CP, _CP), 1)).astype(f32)
    ex = lax.dot_general(sel2f, triu, (((1,), (0,)), ((), ())),
                         preferred_element_type=f32,
                           precision=lax.Precision.HIGHEST)
    chf = (sel2 & (ex == op)).astype(f32)
    vals = jnp.sum(chf * g, axis=1, keepdims=True)
    flatv = jnp.sum(chf * fi2, axis=1, keepdims=True)
    pvalid = pcol < float(_K)
    vals = jnp.where(pvalid, vals, -1.0)
    flatv = jnp.where(pvalid, flatv, float(_N) + pcol)

    eye = (_iotaf((_KP, _KP), 0) ==
           _iotaf((_KP, _KP), 1)).astype(f32)
    vals_row = lax.dot_general(vals, eye, (((0,), (0,)), ((), ())),
                               preferred_element_type=f32,
                           precision=lax.Precision.HIGHEST)
    flat_row = lax.dot_general(flatv, eye, (((0,), (0,)), ((), ())),
                               preferred_element_type=f32,
                           precision=lax.Precision.HIGHEST)
    better = (vals_row > vals) | ((vals_row == vals) & (flat_row < flatv))
    rank = jnp.sum(better.astype(f32), axis=1, keepdims=True)
    scat = (rank == _iotaf((_KP, _KP), 1)).astype(f32)
    scores_row = lax.dot_general(vals, scat, (((0,), (0,)), ((), ())),
                                 preferred_element_type=f32,
                           precision=lax.Precision.HIGHEST)
    flat_sorted_row = lax.dot_general(flatv, scat, (((0,), (0,)), ((), ())),
                                      preferred_element_type=f32,
                           precision=lax.Precision.HIGHEST)
    scores_ref[...] = scores_row[:, :_K].reshape(1, 1, _K)
    labels_ref[...] = (flat_sorted_row.astype(jnp.int32) % _C)[:, :_K].reshape(1, 1, _K)

    flat_sorted_col = lax.dot_general(scat, flatv, (((0,), (0,)), ((), ())),
                                      preferred_element_type=f32,
                           precision=lax.Precision.HIGHEST)
    qs = (flat_sorted_col.astype(jnp.int32) // _C).astype(f32)
    bsel = (qs == _iotaf((_KP, _Q), 1)).astype(f32)
    bx = boxes_ref[...].reshape(_Q, 4)
    cx, cy, bw, bh = bx[:, 0:1], bx[:, 1:2], bx[:, 2:3], bx[:, 3:4]
    hf = ts_ref[0, 0, 0].astype(f32)
    wf = ts_ref[0, 0, 1].astype(f32)
    bscaled = jnp.concatenate(
        [(cx - 0.5 * bw) * wf, (cy - 0.5 * bh) * hf,
         (cx + 0.5 * bw) * wf, (cy + 0.5 * bh) * hf], axis=1)
    boxes_sorted = lax.dot_general(bsel, bscaled, (((1,), (0,)), ((), ())),
                                   preferred_element_type=f32,
                           precision=lax.Precision.HIGHEST)
    boxes_out_ref[...] = boxes_sorted[:_K, :].reshape(1, _K, 4)


def kernel(pred_logits, pred_boxes, positive_map, target_sizes):
    scores, labels, boxes = pl.pallas_call(
        _postprocess_kernel,
        grid=(_B,),
        in_specs=[
            pl.BlockSpec((1, _Q, _T), lambda b: (b, 0, 0)),
            pl.BlockSpec((1, _Q, 4), lambda b: (b, 0, 0)),
            pl.BlockSpec((_C, _T), lambda b: (0, 0)),
            pl.BlockSpec((1, 1, 2), lambda b: (b, 0, 0), memory_space=pltpu.SMEM),
        ],
        out_specs=[
            pl.BlockSpec((1, 1, _K), lambda b: (b, 0, 0)),
            pl.BlockSpec((1, 1, _K), lambda b: (b, 0, 0)),
            pl.BlockSpec((1, _K, 4), lambda b: (b, 0, 0)),
        ],
        out_shape=[
            jax.ShapeDtypeStruct((_B, 1, _K), jnp.float32),
            jax.ShapeDtypeStruct((_B, 1, _K), jnp.int32),
            jax.ShapeDtypeStruct((_B, _K, 4), jnp.float32),
        ],
    )(pred_logits, pred_boxes, positive_map, target_sizes.reshape(_B, 1, 2))
    return scores.reshape(_B, _K), labels.reshape(_B, _K), boxes


# speedup vs baseline: 2.2389x; 1.2819x over previous
import functools
import jax
import jax.numpy as jnp
from jax import lax
from jax.experimental import pallas as pl
from jax.experimental.pallas import tpu as pltpu
from jax.experimental.pallas import tpu_sc as plsc

_B, _Q, _T, _C, _K = 16, 900, 256, 80, 300
_CP = 128
_N = _Q * _C
_ONE_BITS = 0x3F800001
_NW = 32
_HROWS = _Q // 2
_HLEN = _HROWS * _CP
_CAP = 320
_S2 = 2 * _CAP


def _iotaf(shape, dim):
    return lax.broadcasted_iota(jnp.int32, shape, dim).astype(jnp.float32)


def _hp(a, b, dims):
    return lax.dot_general(a, b, (dims, ((), ())),
                           preferred_element_type=jnp.float32,
                           precision=lax.Precision.HIGHEST)


def _tc1_kernel(logits_ref, pmap_ref, prob_ref, vk_ref, j_ref):
    f32 = jnp.float32
    pm = pmap_ref[...]
    sums = jnp.sum(pm, axis=1, keepdims=True)
    safe = jnp.where(sums == 0.0, 1.0, sums)
    pmn = jnp.where(sums != 0.0, pm / safe, pm)
    pmnp = jnp.concatenate([pmn, jnp.zeros((_CP - _C, _T), f32)], axis=0)
    sig = jax.nn.sigmoid(logits_ref[...].reshape(_Q, _T))
    prob = lax.dot_general(sig, pmnp, (((1,), (1,)), ((), ())),
                           preferred_element_type=f32)
    lane = lax.broadcasted_iota(jnp.int32, (_Q, _CP), 1)
    row = lax.broadcasted_iota(jnp.int32, (_Q, _CP), 0)
    w = jnp.where(lane < _C, prob, -1.0)
    fi = row * _C + lane

    def bis1(_, lohi):
        lo, hi = lohi
        mid = lo + (hi - lo) // 2
        t = lax.bitcast_convert_type(mid, f32)
        cnt = jnp.sum((w >= t).astype(f32))
        big = cnt >= float(_K)
        return jnp.where(big, mid, lo), jnp.where(big, hi, mid)

    lo, _hi = lax.fori_loop(0, 31, bis1, (jnp.int32(0), jnp.int32(_ONE_BITS)))
    vk = lax.bitcast_convert_type(lo, f32)
    m = jnp.sum((w > vk).astype(f32))
    r = float(_K) - m

    def bis2(_, lohi):
        lo2, hi2 = lohi
        mid2 = lo2 + (hi2 - lo2) // 2
        cnt2 = jnp.sum(((w == vk) & (fi < mid2)).astype(f32))
        big = cnt2 >= r
        return jnp.where(big, lo2, mid2), jnp.where(big, mid2, hi2)

    _lo2, j = lax.fori_loop(0, 17, bis2, (jnp.int32(0), jnp.int32(_N)))

    prob_ref[...] = w.reshape(1, _Q, _CP)
    vk_ref[...] = jnp.full((1, 2, 16), vk, f32)
    j_ref[...] = jnp.full((1, 2, 16), j, jnp.int32)


def _tc1(pred_logits, positive_map):
    return pl.pallas_call(
        _tc1_kernel,
        grid=(_B,),
        in_specs=[
            pl.BlockSpec((1, _Q, _T), lambda b: (b, 0, 0)),
            pl.BlockSpec((_C, _T), lambda b: (0, 0)),
        ],
        out_specs=[
            pl.BlockSpec((1, _Q, _CP), lambda b: (b, 0, 0)),
            pl.BlockSpec((1, 2, 16), lambda b: (b, 0, 0)),
            pl.BlockSpec((1, 2, 16), lambda b: (b, 0, 0)),
        ],
        out_shape=[
            jax.ShapeDtypeStruct((_B, _Q, _CP), jnp.float32),
            jax.ShapeDtypeStruct((_B, 2, 16), jnp.float32),
            jax.ShapeDtypeStruct((_B, 2, 16), jnp.int32),
        ],
    )(pred_logits, positive_map)


def _sc_compact_kernel(prob_hbm, vk_hbm, j_hbm, svals_hbm, sflat_hbm,
                       pv, valbuf, flatbuf, vkv, jv):
    i32 = jnp.int32
    wid = lax.axis_index("s") * 2 + lax.axis_index("c")
    h = wid % 2
    pltpu.sync_copy(prob_hbm.at[wid], pv)
    pltpu.sync_copy(vk_hbm.at[wid], vkv)
    pltpu.sync_copy(j_hbm.at[wid], jv)
    vkvec = vkv[...]
    jvec = jv[...]
    lanes = lax.broadcasted_iota(i32, (16,), 0)

    def body(q, ptr):
        p = ptr
        for k in range(5):
            v = pv[pl.ds(q * _CP + k * 16, 16)]
            fl = (h * _HROWS + q) * _C + k * 16 + lanes
            msk = (v > vkvec) | ((v == vkvec) & (fl < jvec))
            key = jnp.where(msk, fl, jnp.int32(0x7FFFFFFF))
            _k1, vv = plsc.sort_key_val(key, v)
            _k2, ff = plsc.sort_key_val(key, fl)
            valbuf[pl.ds(p, 16)] = vv
            flatbuf[pl.ds(p, 16)] = ff
            cntv = plsc.all_reduce_population_count(msk)
            p = p + cntv[0]
        return p

    pend = lax.fori_loop(0, _HROWS, body, jnp.int32(0))

    def fill(i, _):
        slot = i * 16 + lanes
        keep = slot < pend
        cv = valbuf[pl.ds(i * 16, 16)]
        cf = flatbuf[pl.ds(i * 16, 16)]
        valbuf[pl.ds(i * 16, 16)] = jnp.where(keep, cv, -1.0)
        flatbuf[pl.ds(i * 16, 16)] = jnp.where(keep, cf, _N + h * _CAP + slot)
        return 0

    lax.fori_loop(0, _CAP // 16, fill, 0)
    pltpu.sync_copy(valbuf, svals_hbm.at[wid])
    pltpu.sync_copy(flatbuf, sflat_hbm.at[wid])


def _sc_compact(prob2, vk2, j2):
    run = pl.kernel(
        _sc_compact_kernel,
        mesh=plsc.VectorSubcoreMesh(core_axis_name="c", subcore_axis_name="s"),
        compiler_params=pltpu.CompilerParams(needs_layout_passes=False),
        out_type=[
            jax.ShapeDtypeStruct((_NW, _CAP), jnp.float32),
            jax.ShapeDtypeStruct((_NW, _CAP), jnp.int32),
        ],
        scratch_types=[
            pltpu.VMEM((_HLEN,), jnp.float32),
            pltpu.VMEM((_CAP,), jnp.float32),
            pltpu.VMEM((_CAP,), jnp.int32),
            pltpu.VMEM((16,), jnp.float32),
            pltpu.VMEM((16,), jnp.int32),
        ],
    )
    return run(prob2, vk2, j2)


def _tc2_kernel(svals_ref, sflat_ref, boxes_ref, ts_ref,
                scores_ref, labels_ref, boxes_out_ref):
    f32 = jnp.float32
    vrow = svals_ref[...].reshape(1, _S2)
    frow = sflat_ref[...].astype(f32).reshape(1, _S2)
    eye = (lax.broadcasted_iota(jnp.int32, (_S2, _S2), 0) ==
           lax.broadcasted_iota(jnp.int32, (_S2, _S2), 1)).astype(f32)
    vcol = _hp(eye, vrow, ((1,), (1,)))
    fcol = _hp(eye, frow, ((1,), (1,)))
    better = (vrow > vcol) | ((vrow == vcol) & (frow < fcol))
    rank = jnp.sum(better.astype(f32), axis=1, keepdims=True)
    scat = (rank == _iotaf((_S2, _S2), 1)).astype(f32)
    scores_row = _hp(vcol, scat, ((0,), (0,)))
    flat_sorted_row = _hp(fcol, scat, ((0,), (0,)))
    scores_ref[...] = scores_row[:, :_K].reshape(1, 1, _K)
    labels_ref[...] = (flat_sorted_row.astype(jnp.int32) % _C)[:, :_K].reshape(1, 1, _K)

    flat_sorted_col = _hp(scat, fcol, ((0,), (0,)))
    qs = (flat_sorted_col.astype(jnp.int32) // _C).astype(f32)
    bsel = (qs == _iotaf((_S2, _Q), 1)).astype(f32)
    bx = boxes_ref[...].reshape(_Q, 4)
    cx, cy, bw, bh = bx[:, 0:1], bx[:, 1:2], bx[:, 2:3], bx[:, 3:4]
    hf = ts_ref[0, 0, 0].astype(f32)
    wf = ts_ref[0, 0, 1].astype(f32)
    bscaled = jnp.concatenate(
        [(cx - 0.5 * bw) * wf, (cy - 0.5 * bh) * hf,
         (cx + 0.5 * bw) * wf, (cy + 0.5 * bh) * hf], axis=1)
    boxes_sorted = _hp(bsel, bscaled, ((1,), (0,)))
    boxes_out_ref[...] = boxes_sorted[:_K, :].reshape(1, _K, 4)


def _tc2(svals, sflat, pred_boxes, ts3):
    return pl.pallas_call(
        _tc2_kernel,
        grid=(_B,),
        in_specs=[
            pl.BlockSpec((1, 1, _S2), lambda b: (b, 0, 0)),
            pl.BlockSpec((1, 1, _S2), lambda b: (b, 0, 0)),
            pl.BlockSpec((1, _Q, 4), lambda b: (b, 0, 0)),
            pl.BlockSpec((1, 1, 2), lambda b: (b, 0, 0), memory_space=pltpu.SMEM),
        ],
        out_specs=[
            pl.BlockSpec((1, 1, _K), lambda b: (b, 0, 0)),
            pl.BlockSpec((1, 1, _K), lambda b: (b, 0, 0)),
            pl.BlockSpec((1, _K, 4), lambda b: (b, 0, 0)),
        ],
        out_shape=[
            jax.ShapeDtypeStruct((_B, 1, _K), jnp.float32),
            jax.ShapeDtypeStruct((_B, 1, _K), jnp.int32),
            jax.ShapeDtypeStruct((_B, _K, 4), jnp.float32),
        ],
    )(svals, sflat, pred_boxes, ts3)


def kernel(pred_logits, pred_boxes, positive_map, target_sizes):
    w_all, vkb, jb = _tc1(pred_logits, positive_map)
    prob2 = w_all.reshape(_NW, _HLEN)
    svals, sflat = _sc_compact(prob2, vkb.reshape(_NW, 16), jb.reshape(_NW, 16))
    scores, labels, boxes = _tc2(svals.reshape(_B, 1, _S2),
                                 sflat.reshape(_B, 1, _S2),
                                 pred_boxes, target_sizes.reshape(_B, 1, 2))
    return scores.reshape(_B, _K), labels.reshape(_B, _K), boxes


# speedup vs baseline: 2.8770x; 1.2850x over previous
import functools
import jax
import jax.numpy as jnp
from jax import lax
from jax.experimental import pallas as pl
from jax.experimental.pallas import tpu as pltpu
from jax.experimental.pallas import tpu_sc as plsc

_B, _Q, _T, _C, _K = 16, 900, 256, 80, 300
_CP = 128
_N = _Q * _C
_ONE_BITS = 0x3F800001
_NW = 32
_HROWS = _Q // 2
_HLEN = _HROWS * _CP
_CAP = 320
_S2 = 2 * _CAP


def _iotaf(shape, dim):
    return lax.broadcasted_iota(jnp.int32, shape, dim).astype(jnp.float32)


def _hp(a, b, dims):
    return lax.dot_general(a, b, (dims, ((), ())),
                           preferred_element_type=jnp.float32,
                           precision=lax.Precision.HIGHEST)


def _tc1_kernel(logits_ref, pmap_ref, prob_ref, vk_ref, j_ref):
    f32 = jnp.float32
    pm = pmap_ref[...]
    sums = jnp.sum(pm, axis=1, keepdims=True)
    safe = jnp.where(sums == 0.0, 1.0, sums)
    pmn = jnp.where(sums != 0.0, pm / safe, pm)
    pmnp = jnp.concatenate([pmn, jnp.zeros((_CP - _C, _T), f32)], axis=0)
    sig = jax.nn.sigmoid(logits_ref[...].reshape(_Q, _T))
    prob = lax.dot_general(sig, pmnp, (((1,), (1,)), ((), ())),
                           preferred_element_type=f32)
    lane = lax.broadcasted_iota(jnp.int32, (_Q, _CP), 1)
    row = lax.broadcasted_iota(jnp.int32, (_Q, _CP), 0)
    w = jnp.where(lane < _C, prob, -1.0)
    fi = row * _C + lane

    def bis1(_, lohi):
        lo, hi = lohi
        mid = lo + (hi - lo) // 2
        t = lax.bitcast_convert_type(mid, f32)
        cnt = jnp.sum((w >= t).astype(f32))
        big = cnt >= float(_K)
        return jnp.where(big, mid, lo), jnp.where(big, hi, mid)

    lo, _hi = lax.fori_loop(0, 31, bis1, (jnp.int32(0), jnp.int32(_ONE_BITS)))
    vk = lax.bitcast_convert_type(lo, f32)
    m = jnp.sum((w > vk).astype(f32))
    r = float(_K) - m

    def bis2(_, lohi):
        lo2, hi2 = lohi
        mid2 = lo2 + (hi2 - lo2) // 2
        cnt2 = jnp.sum(((w == vk) & (fi < mid2)).astype(f32))
        big = cnt2 >= r
        return jnp.where(big, lo2, mid2), jnp.where(big, mid2, hi2)

    total_eq = jnp.sum((w == vk).astype(f32))

    def run_bis2(_):
        _lo2, jv = lax.fori_loop(0, 17, bis2, (jnp.int32(0), jnp.int32(_N)))
        return jv

    j = lax.cond(total_eq > r, run_bis2, lambda _: jnp.int32(_N), None)

    prob_ref[...] = w.reshape(1, _Q, _CP)
    vk_ref[...] = jnp.full((1, 2, 16), vk, f32)
    j_ref[...] = jnp.full((1, 2, 16), j, jnp.int32)


def _tc1(pred_logits, positive_map):
    return pl.pallas_call(
        _tc1_kernel,
        grid=(_B,),
        in_specs=[
            pl.BlockSpec((1, _Q, _T), lambda b: (b, 0, 0)),
            pl.BlockSpec((_C, _T), lambda b: (0, 0)),
        ],
        out_specs=[
            pl.BlockSpec((1, _Q, _CP), lambda b: (b, 0, 0)),
            pl.BlockSpec((1, 2, 16), lambda b: (b, 0, 0)),
            pl.BlockSpec((1, 2, 16), lambda b: (b, 0, 0)),
        ],
        out_shape=[
            jax.ShapeDtypeStruct((_B, _Q, _CP), jnp.float32),
            jax.ShapeDtypeStruct((_B, 2, 16), jnp.float32),
            jax.ShapeDtypeStruct((_B, 2, 16), jnp.int32),
        ],
    )(pred_logits, positive_map)


def _sc_compact_kernel(prob_hbm, vk_hbm, j_hbm, svals_hbm, sflat_hbm,
                       pv, valbuf, flatbuf, vkv, jv):
    i32 = jnp.int32
    wid = lax.axis_index("s") * 2 + lax.axis_index("c")
    h = wid % 2
    pltpu.sync_copy(prob_hbm.at[wid], pv)
    pltpu.sync_copy(vk_hbm.at[wid], vkv)
    pltpu.sync_copy(j_hbm.at[wid], jv)
    vkvec = vkv[...]
    jvec = jv[...]
    lanes = lax.broadcasted_iota(i32, (16,), 0)

    def body(q, ptr):
        p = ptr
        for k in range(5):
            v = pv[pl.ds(q * _CP + k * 16, 16)]
            fl = (h * _HROWS + q) * _C + k * 16 + lanes
            msk = (v > vkvec) | ((v == vkvec) & (fl < jvec))
            key = jnp.where(msk, fl, jnp.int32(0x7FFFFFFF))
            _k1, vv = plsc.sort_key_val(key, v)
            _k2, ff = plsc.sort_key_val(key, fl)
            valbuf[pl.ds(p, 16)] = vv
            flatbuf[pl.ds(p, 16)] = ff
            cntv = plsc.all_reduce_population_count(msk)
            p = p + cntv[0]
        return p

    pend = lax.fori_loop(0, _HROWS, body, jnp.int32(0))

    def fill(i, _):
        slot = i * 16 + lanes
        keep = slot < pend
        cv = valbuf[pl.ds(i * 16, 16)]
        cf = flatbuf[pl.ds(i * 16, 16)]
        valbuf[pl.ds(i * 16, 16)] = jnp.where(keep, cv, -1.0)
        flatbuf[pl.ds(i * 16, 16)] = jnp.where(keep, cf, _N + h * _CAP + slot)
        return 0

    lax.fori_loop(0, _CAP // 16, fill, 0)
    pltpu.sync_copy(valbuf, svals_hbm.at[wid])
    pltpu.sync_copy(flatbuf, sflat_hbm.at[wid])


def _sc_compact(prob2, vk2, j2):
    run = pl.kernel(
        _sc_compact_kernel,
        mesh=plsc.VectorSubcoreMesh(core_axis_name="c", subcore_axis_name="s"),
        compiler_params=pltpu.CompilerParams(needs_layout_passes=False),
        out_type=[
            jax.ShapeDtypeStruct((_NW, _CAP), jnp.float32),
            jax.ShapeDtypeStruct((_NW, _CAP), jnp.int32),
        ],
        scratch_types=[
            pltpu.VMEM((_HLEN,), jnp.float32),
            pltpu.VMEM((_CAP,), jnp.float32),
            pltpu.VMEM((_CAP,), jnp.int32),
            pltpu.VMEM((16,), jnp.float32),
            pltpu.VMEM((16,), jnp.int32),
        ],
    )
    return run(prob2, vk2, j2)


def _tc2_kernel(svals_ref, sflat_ref, boxes_ref, ts_ref,
                scores_ref, labels_ref, boxes_out_ref):
    f32 = jnp.float32
    vrow = svals_ref[...].reshape(1, _S2)
    frow = sflat_ref[...].astype(f32).reshape(1, _S2)
    eye = (lax.broadcasted_iota(jnp.int32, (_S2, _S2), 0) ==
           lax.broadcasted_iota(jnp.int32, (_S2, _S2), 1))
    vcol = jnp.sum(jnp.where(eye, vrow, 0.0), axis=1, keepdims=True)
    fcol = jnp.sum(jnp.where(eye, frow, 0.0), axis=1, keepdims=True)
    better = (vrow > vcol) | ((vrow == vcol) & (frow < fcol))
    rank = jnp.sum(better.astype(f32), axis=1, keepdims=True)
    scat = rank == _iotaf((_S2, _S2), 1)
    scores_row = jnp.sum(jnp.where(scat, vcol, 0.0), axis=0, keepdims=True)
    flat_sorted_row = jnp.sum(jnp.where(scat, fcol, 0.0), axis=0, keepdims=True)
    scores_ref[...] = scores_row[:, :_K].reshape(1, 1, _K)
    labels_ref[...] = (flat_sorted_row.astype(jnp.int32) % _C)[:, :_K].reshape(1, 1, _K)

    flat_sorted_col = jnp.sum(jnp.where(eye, flat_sorted_row, 0.0),
                              axis=1, keepdims=True)
    qs = (flat_sorted_col.astype(jnp.int32) // _C).astype(f32)
    bsel = (qs == _iotaf((_S2, _Q), 1)).astype(f32)
    bx = boxes_ref[...].reshape(_Q, 4)
    cx, cy, bw, bh = bx[:, 0:1], bx[:, 1:2], bx[:, 2:3], bx[:, 3:4]
    hf = ts_ref[0, 0, 0].astype(f32)
    wf = ts_ref[0, 0, 1].astype(f32)
    bscaled = jnp.concatenate(
        [(cx - 0.5 * bw) * wf, (cy - 0.5 * bh) * hf,
         (cx + 0.5 * bw) * wf, (cy + 0.5 * bh) * hf], axis=1)
    boxes_sorted = _hp(bsel, bscaled, ((1,), (0,)))
    boxes_out_ref[...] = boxes_sorted[:_K, :].reshape(1, _K, 4)


def _tc2(svals, sflat, pred_boxes, ts3):
    return pl.pallas_call(
        _tc2_kernel,
        grid=(_B,),
        in_specs=[
            pl.BlockSpec((1, 1, _S2), lambda b: (b, 0, 0)),
            pl.BlockSpec((1, 1, _S2), lambda b: (b, 0, 0)),
            pl.BlockSpec((1, _Q, 4), lambda b: (b, 0, 0)),
            pl.BlockSpec((1, 1, 2), lambda b: (b, 0, 0), memory_space=pltpu.SMEM),
        ],
        out_specs=[
            pl.BlockSpec((1, 1, _K), lambda b: (b, 0, 0)),
            pl.BlockSpec((1, 1, _K), lambda b: (b, 0, 0)),
            pl.BlockSpec((1, _K, 4), lambda b: (b, 0, 0)),
        ],
        out_shape=[
            jax.ShapeDtypeStruct((_B, 1, _K), jnp.float32),
            jax.ShapeDtypeStruct((_B, 1, _K), jnp.int32),
            jax.ShapeDtypeStruct((_B, _K, 4), jnp.float32),
        ],
    )(svals, sflat, pred_boxes, ts3)


def kernel(pred_logits, pred_boxes, positive_map, target_sizes):
    w_all, vkb, jb = _tc1(pred_logits, positive_map)
    prob2 = w_all.reshape(_NW, _HLEN)
    svals, sflat = _sc_compact(prob2, vkb.reshape(_NW, 16), jb.reshape(_NW, 16))
    scores, labels, boxes = _tc2(svals.reshape(_B, 1, _S2),
                                 sflat.reshape(_B, 1, _S2),
                                 pred_boxes, target_sizes.reshape(_B, 1, 2))
    return scores.reshape(_B, _K), labels.reshape(_B, _K), boxes
